# baseline (speedup 1.0000x reference)
import functools

import jax
import jax.numpy as jnp
from jax import lax
from jax.experimental import pallas as pl
from jax.experimental.pallas import tpu as pltpu
from jax.experimental.pallas import tpu_sc as plsc

VOCAB = 32
NPOS = 13
NFPOS = 5
DIM = 128
S, B = 200, 1024
PAD = 1

NC, NS, L = 2, 16, 16
NW = NC * NS
TOTAL = S * B
PER_W = TOTAL // NW
CHUNK = 128
NCHUNK = PER_W // CHUNK
CTAB = VOCAB * NPOS * NFPOS
CTAB_PAD = 2176
ROWS_PER_TILE = CTAB_PAD // NS


def _build_ctab_body(et_ref, pt_ref, ft_ref, out_ref):
    r = lax.broadcasted_iota(jnp.int32, (CTAB_PAD, 1), 0)
    d = r // (NPOS * NFPOS)
    p = (r // NFPOS) % NPOS
    f = r % NFPOS
    cd = lax.broadcasted_iota(jnp.int32, (1, VOCAB), 1)
    cp = lax.broadcasted_iota(jnp.int32, (1, NPOS), 1)
    cf = lax.broadcasted_iota(jnp.int32, (1, NFPOS), 1)
    ohd = ((d == cd) & (d != PAD)).astype(jnp.float32)
    ohp = ((p == cp) & (p != PAD)).astype(jnp.float32)
    ohf = ((f == cf) & (f != PAD)).astype(jnp.float32)
    out_ref[...] = (
        jnp.dot(ohd, et_ref[...], preferred_element_type=jnp.float32)
        + jnp.dot(ohp, pt_ref[...], preferred_element_type=jnp.float32)
        + jnp.dot(ohf, ft_ref[...], preferred_element_type=jnp.float32)
    )


def _build_ctab(et, pt, ft):
    return pl.pallas_call(
        _build_ctab_body,
        out_shape=jax.ShapeDtypeStruct((CTAB_PAD, DIM), jnp.float32),
    )(et, pt, ft)


_MESH = plsc.VectorSubcoreMesh(
    core_axis_name="c", subcore_axis_name="s", num_cores=NC, num_subcores=NS
)


@functools.partial(
    pl.kernel,
    out_type=jax.ShapeDtypeStruct((TOTAL, DIM), jnp.float32),
    mesh=_MESH,
    scratch_types=[
        pltpu.VMEM((PER_W,), jnp.int32),
        pltpu.VMEM((PER_W,), jnp.int32),
        pltpu.VMEM((PER_W,), jnp.int32),
        pltpu.VMEM((NCHUNK, CHUNK), jnp.int32),
        pltpu.VMEM((CHUNK, DIM), jnp.float32),
        pltpu.VMEM((CHUNK, DIM), jnp.float32),
        pltpu.VMEM((CHUNK, DIM), jnp.float32),
        pltpu.VMEM((CHUNK, DIM), jnp.float32),
        pltpu.VMEM_SHARED((CTAB_PAD, DIM), jnp.float32),
        pltpu.SemaphoreType.DMA,
        pltpu.SemaphoreType.DMA,
        pltpu.SemaphoreType.DMA,
        pltpu.SemaphoreType.DMA,
        pltpu.SemaphoreType.DMA,
        pltpu.SemaphoreType.DMA,
        pltpu.SemaphoreType.DMA,
        pltpu.SemaphoreType.DMA,
        pltpu.SemaphoreType.DMA,
    ],
)
def _sc_embed(d_hbm, p_hbm, f_hbm, ctab_hbm, out_hbm,
              d_v, p_v, f_v, idx_v, r0, r1, r2, r3, ctab_sh,
              ps, gs0, gs1, gs2, gs3, ss0, ss1, ss2, ss3):
    sid = lax.axis_index("s")
    wid = sid * NC + lax.axis_index("c")
    base = wid * PER_W
    srow = sid * ROWS_PER_TILE
    stage_cp = pltpu.async_copy(
        ctab_hbm.at[pl.ds(srow, ROWS_PER_TILE)],
        ctab_sh.at[pl.ds(srow, ROWS_PER_TILE)], ps)
    d_cp = pltpu.async_copy(d_hbm.at[pl.ds(base, PER_W)], d_v, ps)
    p_cp = pltpu.async_copy(p_hbm.at[pl.ds(base, PER_W)], p_v, ps)
    f_cp = pltpu.async_copy(f_hbm.at[pl.ds(base, PER_W)], f_v, ps)
    d_cp.wait()
    p_cp.wait()
    f_cp.wait()

    def compute_idx(j):
        for k in range(CHUNK // L):
            off = j * CHUNK + k * L
            d16 = d_v[pl.ds(off, L)]
            p16 = p_v[pl.ds(off, L)]
            f16 = f_v[pl.ds(off, L)]
            idx_v[j, pl.ds(k * L, L)] = d16 * (NPOS * NFPOS) + p16 * NFPOS + f16

    for j in range(6):
        compute_idx(j)
    stage_cp.wait()
    plsc.subcore_barrier()

    bufs = (r0, r1, r2, r3)
    gsems = (gs0, gs1, gs2, gs3)
    ssems = (ss0, ss1, ss2, ss3)

    def g_start(c, b):
        pltpu.async_copy(ctab_sh.at[idx_v.at[c]], bufs[b], gsems[b])

    def g_wait(b):
        pltpu.make_async_copy(ctab_sh.at[idx_v.at[0]], bufs[b], gsems[b]).wait()

    def s_start(c, b):
        pltpu.async_copy(bufs[b], out_hbm.at[pl.ds(base + c * CHUNK, CHUNK)],
                         ssems[b])

    def s_wait(b):
        pltpu.make_async_copy(bufs[b], out_hbm.at[pl.ds(base, CHUNK)],
                              ssems[b]).wait()

    g_start(0, 0)
    g_start(1, 1)
    g_start(2, 2)
    g_wait(0)
    s_start(0, 0)
    g_start(3, 3)
    g_wait(1)
    s_start(1, 1)
    s_wait(0)
    g_start(4, 0)
    g_wait(2)
    s_start(2, 2)
    s_wait(1)
    g_start(5, 1)
    g_wait(3)
    s_start(3, 3)

    def pipelined(t, carry):
        c = 4 * t
        for k in range(4):
            bl = (k + 2) % 4
            g_wait(k)
            s_start(c + k, k)
            compute_idx(c + k + 2)
            s_wait(bl)
            g_start(c + k + 2, bl)
        return carry

    lax.fori_loop(1, NCHUNK // 4, pipelined, 0)

    g_wait(0)
    s_start(NCHUNK - 2, 0)
    g_wait(1)
    s_start(NCHUNK - 1, 1)
    s_wait(2)
    s_wait(3)
    s_wait(0)
    s_wait(1)


def kernel(batch_datasets, batch_positionals, batch_float_positionals,
           emb_table, pos_table, fpos_table):
    ctab = _build_ctab(emb_table, pos_table, fpos_table)
    d = batch_datasets.reshape(-1)
    p = batch_positionals.reshape(-1)
    f = batch_float_positionals.reshape(-1)
    out = _sc_embed(d, p, f, ctab)
    return out.reshape(S, B, DIM)


# speedup vs baseline: 1.0108x; 1.0108x over previous
import functools

import jax
import jax.numpy as jnp
from jax import lax
from jax.experimental import pallas as pl
from jax.experimental.pallas import tpu as pltpu
from jax.experimental.pallas import tpu_sc as plsc

VOCAB = 32
NPOS = 13
NFPOS = 5
DIM = 128
S, B = 200, 1024
PAD = 1

NC, NS, L = 2, 16, 16
NW = NC * NS
TOTAL = S * B
PER_W = TOTAL // NW
CHUNK = 128
NCHUNK = PER_W // CHUNK
CTAB = VOCAB * NPOS * NFPOS
CTAB_PAD = 2176
ROWS_PER_TILE = CTAB_PAD // NS


def _build_ctab_body(et_ref, pt_ref, ft_ref, out_ref):
    r = lax.broadcasted_iota(jnp.int32, (CTAB_PAD, 1), 0)
    d = r // (NPOS * NFPOS)
    p = (r // NFPOS) % NPOS
    f = r % NFPOS
    cd = lax.broadcasted_iota(jnp.int32, (1, VOCAB), 1)
    cp = lax.broadcasted_iota(jnp.int32, (1, NPOS), 1)
    cf = lax.broadcasted_iota(jnp.int32, (1, NFPOS), 1)
    ohd = ((d == cd) & (d != PAD)).astype(jnp.float32)
    ohp = ((p == cp) & (p != PAD)).astype(jnp.float32)
    ohf = ((f == cf) & (f != PAD)).astype(jnp.float32)
    out_ref[...] = (
        jnp.dot(ohd, et_ref[...], preferred_element_type=jnp.float32)
        + jnp.dot(ohp, pt_ref[...], preferred_element_type=jnp.float32)
        + jnp.dot(ohf, ft_ref[...], preferred_element_type=jnp.float32)
    )


def _build_ctab(et, pt, ft):
    return pl.pallas_call(
        _build_ctab_body,
        out_shape=jax.ShapeDtypeStruct((CTAB_PAD, DIM), jnp.float32),
    )(et, pt, ft)


_MESH = plsc.VectorSubcoreMesh(
    core_axis_name="c", subcore_axis_name="s", num_cores=NC, num_subcores=NS
)


@functools.partial(
    pl.kernel,
    out_type=jax.ShapeDtypeStruct((TOTAL, DIM), jnp.float32),
    mesh=_MESH,
    scratch_types=[
        pltpu.VMEM((PER_W,), jnp.int32),
        pltpu.VMEM((PER_W,), jnp.int32),
        pltpu.VMEM((PER_W,), jnp.int32),
        pltpu.VMEM((NCHUNK, CHUNK), jnp.int32),
        pltpu.VMEM((CHUNK, DIM), jnp.float32),
        pltpu.VMEM((CHUNK, DIM), jnp.float32),
        pltpu.VMEM((CHUNK, DIM), jnp.float32),
        pltpu.VMEM((CHUNK, DIM), jnp.float32),
        pltpu.VMEM_SHARED((CTAB_PAD, DIM), jnp.float32),
        pltpu.SemaphoreType.DMA,
        pltpu.SemaphoreType.DMA,
        pltpu.SemaphoreType.DMA,
        pltpu.SemaphoreType.DMA,
        pltpu.SemaphoreType.DMA,
        pltpu.SemaphoreType.DMA,
        pltpu.SemaphoreType.DMA,
        pltpu.SemaphoreType.DMA,
        pltpu.SemaphoreType.DMA,
    ],
)
def _sc_embed(d_hbm, p_hbm, f_hbm, ctab_hbm, out_hbm,
              d_v, p_v, f_v, idx_v, r0, r1, r2, r3, ctab_sh,
              ps, gs0, gs1, gs2, gs3, ss0, ss1, ss2, ss3):
    sid = lax.axis_index("s")
    wid = sid * NC + lax.axis_index("c")
    base = wid * PER_W
    srow = sid * ROWS_PER_TILE
    stage_cp = pltpu.async_copy(
        ctab_hbm.at[pl.ds(srow, ROWS_PER_TILE)],
        ctab_sh.at[pl.ds(srow, ROWS_PER_TILE)], ps)
    d_cp = pltpu.async_copy(d_hbm.at[pl.ds(base, PER_W)], d_v, ps)
    p_cp = pltpu.async_copy(p_hbm.at[pl.ds(base, PER_W)], p_v, ps)
    f_cp = pltpu.async_copy(f_hbm.at[pl.ds(base, PER_W)], f_v, ps)
    d_cp.wait()
    p_cp.wait()
    f_cp.wait()

    def compute_idx(j):
        for k in range(CHUNK // L):
            off = j * CHUNK + k * L
            d16 = d_v[pl.ds(off, L)]
            p16 = p_v[pl.ds(off, L)]
            f16 = f_v[pl.ds(off, L)]
            idx_v[j, pl.ds(k * L, L)] = d16 * (NPOS * NFPOS) + p16 * NFPOS + f16

    for j in range(6):
        compute_idx(j)
    stage_cp.wait()
    plsc.subcore_barrier()

    bufs = (r0, r1, r2, r3)
    gsems = (gs0, gs1, gs2, gs3)
    ssems = (ss0, ss1, ss2, ss3)

    def g_start(c, b):
        pltpu.async_copy(ctab_sh.at[idx_v.at[c]], bufs[b], gsems[b])

    def g_wait(b):
        pltpu.make_async_copy(ctab_sh.at[idx_v.at[0]], bufs[b], gsems[b]).wait()

    def s_start(c, b):
        pltpu.async_copy(bufs[b], out_hbm.at[pl.ds(base + c * CHUNK, CHUNK)],
                         ssems[b])

    def s_wait(b):
        pltpu.make_async_copy(bufs[b], out_hbm.at[pl.ds(base, CHUNK)],
                              ssems[b]).wait()

    g_start(0, 0)
    g_start(1, 1)
    g_start(2, 2)
    g_wait(0)
    s_start(0, 0)
    g_start(3, 3)
    g_wait(1)
    s_start(1, 1)
    s_wait(0)
    g_start(4, 0)
    g_wait(2)
    s_start(2, 2)
    s_wait(1)
    g_start(5, 1)
    g_wait(3)
    s_start(3, 3)

    def pipelined(t, carry):
        c = 4 * t
        for k in range(4):
            bl = (k + 2) % 4
            compute_idx(c + k + 2)
            s_wait(bl)
            g_start(c + k + 2, bl)
            g_wait(k)
            s_start(c + k, k)
        return carry

    lax.fori_loop(1, NCHUNK // 4, pipelined, 0)

    g_wait(0)
    s_start(NCHUNK - 2, 0)
    g_wait(1)
    s_start(NCHUNK - 1, 1)
    s_wait(2)
    s_wait(3)
    s_wait(0)
    s_wait(1)


def kernel(batch_datasets, batch_positionals, batch_float_positionals,
           emb_table, pos_table, fpos_table):
    ctab = _build_ctab(emb_table, pos_table, fpos_table)
    d = batch_datasets.reshape(-1)
    p = batch_positionals.reshape(-1)
    f = batch_float_positionals.reshape(-1)
    out = _sc_embed(d, p, f, ctab)
    return out.reshape(S, B, DIM)
